# baseline (speedup 1.0000x reference)
import functools

import jax
import jax.numpy as jnp
from jax import lax
from jax.experimental import pallas as pl
from jax.experimental.pallas import tpu as pltpu
from jax.experimental.pallas import tpu_sc as plsc

D = 128
DH = D // 2
DW = 64
TAU = 0.5
NC = 2
NS = 16
CH = 128
NPASS = 4
NBUF = 2
LANES = 16


def _sc_accumulate(x_halves, srcp, dstp, tp, n_pad, nchunk):
  rows_pt = n_pad // NS
  nzc = rows_pt // CH
  mesh = plsc.VectorSubcoreMesh(core_axis_name="c", subcore_axis_name="s")

  ncp = nchunk // NPASS
  @functools.partial(
      pl.kernel,
      out_type=(jax.ShapeDtypeStruct((NC, n_pad, DW), jnp.float32),
                jax.ShapeDtypeStruct((NC, NS, n_pad), jnp.float32)),
      mesh=mesh,
      scratch_types=[
          pltpu.VMEM((ncp, CH), jnp.int32),
          pltpu.VMEM((ncp, CH), jnp.int32),
          pltpu.VMEM((ncp, CH), jnp.float32),
          [pltpu.VMEM((CH, DW), jnp.float32)] * NBUF,
          pltpu.VMEM((n_pad,), jnp.float32),
          pltpu.VMEM_SHARED((n_pad, DW), jnp.float32),
          pltpu.VMEM_SHARED((n_pad, DW), jnp.float32),
          [pltpu.SemaphoreType.DMA] * NBUF,
          [pltpu.SemaphoreType.DMA] * NBUF,
      ],
      compiler_params=pltpu.CompilerParams(
          use_tc_tiling_on_sc=False, needs_layout_passes=False),
  )
  def body(x_hbm, src_hbm, dst_hbm, t_hbm, out_hbm, den_hbm,
           src_v, dst_v, w_v, rows, den_v, x_sh, acc_sh, gsems, ssems):
    c = lax.axis_index("c")
    s = lax.axis_index("s")

    def stg(k, carry):
      pltpu.sync_copy(
          x_hbm.at[c, pl.ds(s * rows_pt + k * CH, CH)],
          x_sh.at[pl.ds(s * rows_pt + k * CH, CH)])
      return carry
    lax.fori_loop(0, nzc, stg, 0)

    zero = jnp.zeros((LANES,), jnp.float32)

    def zden(i, carry):
      den_v[pl.ds(i * LANES, LANES)] = zero
      return carry
    lax.fori_loop(0, n_pad // LANES, zden, 0)

    def zrow(i, carry):
      for q in range(DW // LANES):
        rows[0][i, pl.ds(q * LANES, LANES)] = zero
      return carry
    lax.fori_loop(0, CH, zrow, 0)

    def zcp(k, carry):
      pltpu.sync_copy(
          rows[0],
          acc_sh.at[pl.ds(s * rows_pt + k * CH, CH)])
      return carry
    lax.fori_loop(0, nzc, zcp, 0)
    plsc.subcore_barrier()

    inv_tau = jnp.float32(1.0 / (TAU + 1e-8))

    def scale(rows_b, j):
      def sgrp(g, carry2):
        w16 = w_v[j, pl.ds(g * LANES, LANES)]
        for l in range(LANES):
          w = w16[l]
          r = g * LANES + l
          for q in range(DW // LANES):
            v = rows_b[r, pl.ds(q * LANES, LANES)]
            rows_b[r, pl.ds(q * LANES, LANES)] = v * w
        return carry2
      lax.fori_loop(0, CH // LANES, sgrp, 0)

    def gstart(b, j):
      pltpu.async_copy(x_sh.at[src_v.at[j]], rows[b], gsems[b])

    def gwait(b):
      pltpu.make_async_copy(x_sh.at[src_v.at[0]], rows[b], gsems[b]).wait()

    def sstart(b, j):
      pltpu.async_copy(rows[b], acc_sh.at[dst_v.at[j]], ssems[b], add=True)

    def swait(b):
      pltpu.make_async_copy(rows[b], acc_sh.at[dst_v.at[0]], ssems[b]).wait()

    for p in range(NPASS):
      base = s * nchunk + p * ncp
      pltpu.sync_copy(src_hbm.at[pl.ds(base, ncp)], src_v)
      pltpu.sync_copy(dst_hbm.at[pl.ds(base, ncp)], dst_v)
      pltpu.sync_copy(t_hbm.at[pl.ds(base, ncp)], w_v)

      def wrow(i, carry):
        for q in range(CH // LANES):
          t16 = w_v[i, pl.ds(q * LANES, LANES)]
          w_v[i, pl.ds(q * LANES, LANES)] = jnp.exp(t16 * inv_tau)
        return carry
      lax.fori_loop(0, ncp, wrow, 0)

      def drow(i, carry):
        for q in range(CH // LANES):
          idx16 = dst_v[i, pl.ds(q * LANES, LANES)]
          w16 = w_v[i, pl.ds(q * LANES, LANES)]
          plsc.addupdate_scatter(den_v, [idx16], w16)
        return carry
      lax.fori_loop(0, ncp, drow, 0)

      gstart(0, 0)

      def pair(h2, carry):
        j = h2 * 2

        @pl.when(h2 > 0)
        def _():
          swait(1)
        gstart(1, j + 1)
        gwait(0)
        scale(rows[0], j)
        sstart(0, j)

        @pl.when(j + 2 < ncp)
        def _():
          swait(0)
          gstart(0, j + 2)
        gwait(1)
        scale(rows[1], j + 1)
        sstart(1, j + 1)
        return carry
      lax.fori_loop(0, ncp // 2, pair, 0)

      swait(0)
      swait(1)

    plsc.subcore_barrier()
    pltpu.sync_copy(
        acc_sh.at[pl.ds(s * rows_pt, rows_pt)],
        out_hbm.at[c, pl.ds(s * rows_pt, rows_pt)])
    pltpu.sync_copy(den_v, den_hbm.at[c, s])

  return body(x_halves, srcp, dstp, tp)


def _tc_finish(partial, dens, w_t, n_nodes):
  blk = 1000

  def body(p_ref, d_ref, wt_ref, out_ref):
    den = jnp.sum(d_ref[...], axis=1)[:, None]
    inv = jnp.float32(1.0) / (den + jnp.float32(1e-16))
    s0 = p_ref[0] * inv
    s1 = p_ref[1] * inv
    out_ref[...] = (
        jnp.dot(s0, wt_ref[:DH, :], preferred_element_type=jnp.float32)
        + jnp.dot(s1, wt_ref[DH:, :], preferred_element_type=jnp.float32))

  return pl.pallas_call(
      body,
      grid=(n_nodes // blk,),
      in_specs=[
          pl.BlockSpec((NC, blk, DW), lambda i: (0, i, 0)),
          pl.BlockSpec((blk, NS), lambda i: (i, 0)),
          pl.BlockSpec((D, D), lambda i: (0, 0)),
      ],
      out_specs=pl.BlockSpec((blk, D), lambda i: (i, 0)),
      out_shape=jax.ShapeDtypeStruct((n_nodes, D), jnp.float32),
  )(partial, dens, w_t)


def kernel(x_src, x_dst, edge_index, edge_attr_time, W):
  n_nodes = x_src.shape[0]
  n_pad = ((n_nodes + NS * CH - 1) // (NS * CH)) * (NS * CH)
  e = edge_index.shape[1]
  e_pw = NS * CH * 8
  e_pad = ((e + e_pw - 1) // e_pw) * e_pw
  nchunk = e_pad // (NS * CH)
  pad = e_pad - e

  src = edge_index[0]
  dst = edge_index[1]
  t = edge_attr_time.reshape(-1).astype(jnp.float32)
  srcp = jnp.concatenate(
      [src, jnp.zeros((pad,), jnp.int32)]).reshape(e_pad // CH, CH)
  dstp = jnp.concatenate(
      [dst, jnp.zeros((pad,), jnp.int32)]).reshape(e_pad // CH, CH)
  tp = jnp.concatenate(
      [t, jnp.full((pad,), -100.0, jnp.float32)]).reshape(e_pad // CH, CH)
  xf = x_src.astype(jnp.float32)
  rpad = jnp.zeros((n_pad - n_nodes, DW), jnp.float32)
  x_halves = jnp.stack(
      [jnp.concatenate([xf[:, :DH], rpad]),
       jnp.concatenate([xf[:, DH:], rpad])])

  partial, dens = _sc_accumulate(x_halves, srcp, dstp, tp, n_pad, nchunk)
  dens0 = dens[0].T
  return _tc_finish(partial, dens0, W.astype(jnp.float32).T, n_nodes)


# speedup vs baseline: 1.8186x; 1.8186x over previous
import functools

import jax
import jax.numpy as jnp
from jax import lax
from jax.experimental import pallas as pl
from jax.experimental.pallas import tpu as pltpu
from jax.experimental.pallas import tpu_sc as plsc

D = 128
DH = D // 2
DW = 80
TAU = 0.5
NC = 2
NS = 16
CH = 64
NPASS = 8
NBUF = 4
LANES = 16


def _sc_accumulate(x_halves, srcp, dstp, tp, n_pad, nchunk):
  rows_pt = n_pad // NS
  nzc = rows_pt // CH
  mesh = plsc.VectorSubcoreMesh(core_axis_name="c", subcore_axis_name="s")

  ncp = nchunk // NPASS
  @functools.partial(
      pl.kernel,
      out_type=jax.ShapeDtypeStruct((NC, n_pad, DW), jnp.float32),
      mesh=mesh,
      scratch_types=[
          pltpu.VMEM((ncp, CH), jnp.int32),
          pltpu.VMEM((ncp, CH), jnp.int32),
          pltpu.VMEM((ncp, CH), jnp.float32),
          [pltpu.VMEM((CH, DW), jnp.float32)] * NBUF,
          pltpu.VMEM_SHARED((n_pad, DW), jnp.float32),
          pltpu.VMEM_SHARED((n_pad, DW), jnp.float32),
          [pltpu.SemaphoreType.DMA] * NBUF,
          [pltpu.SemaphoreType.DMA] * NBUF,
      ],
      compiler_params=pltpu.CompilerParams(use_tc_tiling_on_sc=False),
  )
  def body(x_hbm, src_hbm, dst_hbm, t_hbm, out_hbm,
           src_v, dst_v, w_v, rows, x_sh, acc_sh, gsems, ssems):
    c = lax.axis_index("c")
    s = lax.axis_index("s")

    def stg(k, carry):
      pltpu.sync_copy(
          x_hbm.at[c, pl.ds(s * rows_pt + k * CH, CH)],
          x_sh.at[pl.ds(s * rows_pt + k * CH, CH)])
      return carry
    lax.fori_loop(0, nzc, stg, 0)

    zero = jnp.zeros((LANES,), jnp.float32)

    def zrow(i, carry):
      for q in range(DW // LANES):
        rows[0][i, pl.ds(q * LANES, LANES)] = zero
      return carry
    lax.fori_loop(0, CH, zrow, 0)

    def zcp(k, carry):
      pltpu.sync_copy(
          rows[0],
          acc_sh.at[pl.ds(s * rows_pt + k * CH, CH)])
      return carry
    lax.fori_loop(0, nzc, zcp, 0)
    plsc.subcore_barrier()

    inv_tau = jnp.float32(1.0 / (TAU + 1e-8))

    def scale(rows_b, j):
      def sgrp(g, carry2):
        w16 = w_v[j, pl.ds(g * LANES, LANES)]
        for l in range(LANES):
          w = w16[l]
          r = g * LANES + l
          for q in range(DW // LANES):
            v = rows_b[r, pl.ds(q * LANES, LANES)]
            rows_b[r, pl.ds(q * LANES, LANES)] = v * w
        return carry2
      lax.fori_loop(0, CH // LANES, sgrp, 0)

    def gstart(b, j):
      pltpu.async_copy(x_sh.at[src_v.at[j]], rows[b], gsems[b])

    def gwait(b):
      pltpu.make_async_copy(x_sh.at[src_v.at[0]], rows[b], gsems[b]).wait()

    def sstart(b, j):
      pltpu.async_copy(rows[b], acc_sh.at[dst_v.at[j]], ssems[b], add=True)

    def swait(b):
      pltpu.make_async_copy(rows[b], acc_sh.at[dst_v.at[0]], ssems[b]).wait()

    for p in range(NPASS):
      base = s * nchunk + p * ncp
      pltpu.sync_copy(src_hbm.at[pl.ds(base, ncp)], src_v)
      pltpu.sync_copy(dst_hbm.at[pl.ds(base, ncp)], dst_v)
      pltpu.sync_copy(t_hbm.at[pl.ds(base, ncp)], w_v)

      def wrow(i, carry):
        for q in range(CH // LANES):
          t16 = w_v[i, pl.ds(q * LANES, LANES)]
          w_v[i, pl.ds(q * LANES, LANES)] = jnp.exp(t16 * inv_tau)
        return carry
      lax.fori_loop(0, ncp, wrow, 0)

      gstart(0, 0)
      gstart(1, 1)

      def quad(q, carry):
        j0 = q * NBUF
        for r in range(NBUF):
          j = j0 + r
          m = j + 2
          bm = (r + 2) % NBUF

          if r < 2:
            @pl.when(q > 0)
            def _():
              swait(bm)
            gstart(bm, m)
          else:
            swait(bm)

            @pl.when(m < ncp)
            def _():
              gstart(bm, m)

          gwait(r)
          scale(rows[r], j)
          sstart(r, j)
        return carry
      lax.fori_loop(0, ncp // NBUF, quad, 0)

      swait((ncp - 2) % NBUF)
      swait((ncp - 1) % NBUF)

    plsc.subcore_barrier()
    pltpu.sync_copy(
        acc_sh.at[pl.ds(s * rows_pt, rows_pt)],
        out_hbm.at[c, pl.ds(s * rows_pt, rows_pt)])

  return body(x_halves, srcp, dstp, tp)


def _tc_finish(partial, w_t, n_nodes):
  blk = 1000

  def body(p_ref, wt_ref, out_ref):
    a0 = p_ref[0]
    a1 = p_ref[1]
    eps = jnp.float32(1e-16)
    s0 = a0[:, :DH] / (a0[:, DH:DH + 1] + eps)
    s1 = a1[:, :DH] / (a1[:, DH:DH + 1] + eps)
    out_ref[...] = (
        jnp.dot(s0, wt_ref[:DH, :], preferred_element_type=jnp.float32)
        + jnp.dot(s1, wt_ref[DH:, :], preferred_element_type=jnp.float32))

  return pl.pallas_call(
      body,
      grid=(n_nodes // blk,),
      in_specs=[
          pl.BlockSpec((NC, blk, DW), lambda i: (0, i, 0)),
          pl.BlockSpec((D, D), lambda i: (0, 0)),
      ],
      out_specs=pl.BlockSpec((blk, D), lambda i: (i, 0)),
      out_shape=jax.ShapeDtypeStruct((n_nodes, D), jnp.float32),
  )(partial, w_t)


def kernel(x_src, x_dst, edge_index, edge_attr_time, W):
  n_nodes = x_src.shape[0]
  n_pad = ((n_nodes + NS * CH - 1) // (NS * CH)) * (NS * CH)
  e = edge_index.shape[1]
  e_pw = NS * CH * 8
  e_pad = ((e + e_pw - 1) // e_pw) * e_pw
  nchunk = e_pad // (NS * CH)
  pad = e_pad - e

  src = edge_index[0]
  dst = edge_index[1]
  t = edge_attr_time.reshape(-1).astype(jnp.float32)
  srcp = jnp.concatenate(
      [src, jnp.zeros((pad,), jnp.int32)]).reshape(e_pad // CH, CH)
  dstp = jnp.concatenate(
      [dst, jnp.zeros((pad,), jnp.int32)]).reshape(e_pad // CH, CH)
  tp = jnp.concatenate(
      [t, jnp.full((pad,), -100.0, jnp.float32)]).reshape(e_pad // CH, CH)
  xf = x_src.astype(jnp.float32)
  ones = jnp.ones((n_nodes, 1), jnp.float32)
  zpad = jnp.zeros((n_nodes, DW - DH - 1), jnp.float32)
  rpad = jnp.zeros((n_pad - n_nodes, DW), jnp.float32)
  x_halves = jnp.stack(
      [jnp.concatenate(
          [jnp.concatenate([xf[:, :DH], ones, zpad], axis=1), rpad]),
       jnp.concatenate(
          [jnp.concatenate([xf[:, DH:], ones, zpad], axis=1), rpad])])

  partial = _sc_accumulate(x_halves, srcp, dstp, tp, n_pad, nchunk)
  return _tc_finish(partial, W.astype(jnp.float32).T, n_nodes)
